# Initial kernel scaffold; baseline (speedup 1.0000x reference)
import functools

import jax
import jax.numpy as jnp
from jax import lax
from jax.experimental import pallas as pl
from jax.experimental.pallas import tpu as pltpu
from jax.experimental.pallas import tpu_sc as plsc

N_NODES = 10000
D_FEAT = 128
N_EDGES = 320000

NC = 2
NS = 16
NW = NC * NS
CHUNK = 128
CHUNKS_PER_W = 80
EDGES_PER_W = CHUNK * CHUNKS_PER_W
E_PAD = NW * EDGES_PER_W
ROWS_PER_W = N_NODES // NS
DL = D_FEAT // 16


def _sc_body(x_hbm, row_hbm, col_hbm, w_hbm, out_hbm,
             row_all, col_all, w_all, rows_v, shared, sem):
    c = lax.axis_index("c")
    s = lax.axis_index("s")
    wid = s * NC + c

    def zero_body(e, _):
        for d in range(DL):
            rows_v[e, pl.ds(d * 16, 16)] = jnp.zeros((16,), jnp.float32)
        return 0
    lax.fori_loop(0, CHUNK, zero_body, 0)
    base = s * ROWS_PER_W
    for k in range(4):
        pltpu.sync_copy(rows_v, shared.at[pl.ds(base + k * CHUNK, CHUNK)])
    pltpu.sync_copy(rows_v.at[pl.ds(0, ROWS_PER_W - 4 * CHUNK)],
                    shared.at[pl.ds(base + 4 * CHUNK, ROWS_PER_W - 4 * CHUNK)])

    pltpu.sync_copy(row_hbm.at[wid], row_all)
    pltpu.sync_copy(col_hbm.at[wid], col_all)
    pltpu.sync_copy(w_hbm.at[wid], w_all)

    def clean_body(j, _):
        for d in range(DL):
            wv = w_all[j, pl.ds(d * 16, 16)]
            w_all[j, pl.ds(d * 16, 16)] = jnp.where((wv - wv) == 0.0, wv, 0.0)
        return 0
    lax.fori_loop(0, CHUNKS_PER_W, clean_body, 0)

    plsc.subcore_barrier()

    def chunk_body(j, _):
        pltpu.async_copy(x_hbm.at[row_all.at[j]], rows_v, sem).wait()

        def mul_body(e, _2):
            w = w_all[j, e]
            for d in range(DL):
                sl = rows_v[e, pl.ds(d * 16, 16)]
                rows_v[e, pl.ds(d * 16, 16)] = sl * w
            return 0
        lax.fori_loop(0, CHUNK, mul_body, 0)

        pltpu.sync_copy(rows_v, shared.at[col_all.at[j]], add=True)
        return 0
    lax.fori_loop(0, CHUNKS_PER_W, chunk_body, 0)

    plsc.subcore_barrier()

    pltpu.sync_copy(shared.at[pl.ds(base, ROWS_PER_W)],
                    out_hbm.at[c, pl.ds(base, ROWS_PER_W)])


_sc_spmm = functools.partial(
    pl.kernel,
    out_type=jax.ShapeDtypeStruct((NC, N_NODES, D_FEAT), jnp.float32),
    mesh=plsc.VectorSubcoreMesh(core_axis_name="c", subcore_axis_name="s"),
    scratch_types=[
        pltpu.VMEM((CHUNKS_PER_W, CHUNK), jnp.int32),
        pltpu.VMEM((CHUNKS_PER_W, CHUNK), jnp.int32),
        pltpu.VMEM((CHUNKS_PER_W, CHUNK), jnp.float32),
        pltpu.VMEM((CHUNK, D_FEAT), jnp.float32),
        pltpu.VMEM_SHARED((N_NODES, D_FEAT), jnp.float32),
        pltpu.SemaphoreType.DMA,
    ],
)(_sc_body)


def _add_body(p_ref, o_ref):
    o_ref[...] = p_ref[0] + p_ref[1]


def _sum_partials(p):
    bn = 1000
    return pl.pallas_call(
        _add_body,
        grid=(N_NODES // bn,),
        in_specs=[pl.BlockSpec((NC, bn, D_FEAT), lambda i: (0, i, 0))],
        out_specs=pl.BlockSpec((bn, D_FEAT), lambda i: (i, 0)),
        out_shape=jax.ShapeDtypeStruct((N_NODES, D_FEAT), jnp.float32),
    )(p)


def kernel(inputs, edge_index, edge_weight):
    row = edge_index[0].astype(jnp.int32)
    col = edge_index[1].astype(jnp.int32)
    w = edge_weight.astype(jnp.float32)
    pad = E_PAD - row.shape[0]
    row = jnp.pad(row, (0, pad)).reshape(NW, CHUNKS_PER_W, CHUNK)
    col = jnp.pad(col, (0, pad)).reshape(NW, CHUNKS_PER_W, CHUNK)
    w = jnp.pad(w, (0, pad)).reshape(NW, CHUNKS_PER_W, CHUNK)
    partials = _sc_spmm(inputs, row, col, w)
    return _sum_partials(partials)


# speedup vs baseline: 3.0643x; 3.0643x over previous
import functools

import jax
import jax.numpy as jnp
from jax import lax
from jax.experimental import pallas as pl
from jax.experimental.pallas import tpu as pltpu
from jax.experimental.pallas import tpu_sc as plsc

N_NODES = 10000
D_FEAT = 128
N_EDGES = 320000

NC = 2
NS = 16
NW = NC * NS
CHUNK = 128
CHUNKS_PER_W = 80
EDGES_PER_W = CHUNK * CHUNKS_PER_W
E_PAD = NW * EDGES_PER_W
N_PAD = 10112
ROWS_PER_W = N_PAD // NS
DL = D_FEAT // 16


def _sc_body(x_hbm, row_hbm, col_hbm, w_hbm, out_hbm,
             row_all, col_all, w_all, rows_v, shared, sem):
    c = lax.axis_index("c")
    s = lax.axis_index("s")
    wid = s * NC + c

    def zero_body(e, _):
        for d in range(DL):
            rows_v[e, pl.ds(d * 16, 16)] = jnp.zeros((16,), jnp.float32)
        return 0
    lax.fori_loop(0, CHUNK, zero_body, 0)
    base = s * ROWS_PER_W
    for k in range(4):
        pltpu.sync_copy(rows_v, shared.at[pl.ds(base + k * CHUNK, CHUNK)])
    pltpu.sync_copy(rows_v.at[pl.ds(0, ROWS_PER_W - 4 * CHUNK)],
                    shared.at[pl.ds(base + 4 * CHUNK, ROWS_PER_W - 4 * CHUNK)])

    pltpu.sync_copy(row_hbm.at[wid], row_all)
    pltpu.sync_copy(col_hbm.at[wid], col_all)
    pltpu.sync_copy(w_hbm.at[wid], w_all)

    def clean_body(j, _):
        for d in range(DL):
            wv = w_all[j, pl.ds(d * 16, 16)]
            w_all[j, pl.ds(d * 16, 16)] = jnp.where((wv - wv) == 0.0, wv, 0.0)
        return 0
    lax.fori_loop(0, CHUNKS_PER_W, clean_body, 0)

    plsc.subcore_barrier()

    def chunk_body(j, _):
        pltpu.async_copy(x_hbm.at[row_all.at[j]], rows_v, sem).wait()

        def mul_body(g, _2):
            wv = w_all[j, pl.ds(g * 16, 16)]
            for l in range(16):
                w = wv[l]
                e = g * 16 + l
                for d in range(DL):
                    sl = rows_v[e, pl.ds(d * 16, 16)]
                    rows_v[e, pl.ds(d * 16, 16)] = sl * w
            return 0
        lax.fori_loop(0, CHUNK // 16, mul_body, 0)

        pltpu.sync_copy(rows_v, shared.at[col_all.at[j]], add=True)
        return 0
    lax.fori_loop(0, CHUNKS_PER_W, chunk_body, 0)

    plsc.subcore_barrier()

    pltpu.sync_copy(shared.at[pl.ds(base, ROWS_PER_W)],
                    out_hbm.at[c, pl.ds(base, ROWS_PER_W)])


_sc_spmm = functools.partial(
    pl.kernel,
    out_type=jax.ShapeDtypeStruct((NC, N_PAD, D_FEAT), jnp.float32),
    mesh=plsc.VectorSubcoreMesh(core_axis_name="c", subcore_axis_name="s"),
    scratch_types=[
        pltpu.VMEM((CHUNKS_PER_W, CHUNK), jnp.int32),
        pltpu.VMEM((CHUNKS_PER_W, CHUNK), jnp.int32),
        pltpu.VMEM((CHUNKS_PER_W, CHUNK), jnp.float32),
        pltpu.VMEM((CHUNK, D_FEAT), jnp.float32),
        pltpu.VMEM_SHARED((N_PAD, D_FEAT), jnp.float32),
        pltpu.SemaphoreType.DMA,
    ],
)(_sc_body)


def _add_body(p_ref, o_ref):
    o_ref[...] = p_ref[0] + p_ref[1]


def _sum_partials(p):
    bn = ROWS_PER_W
    return pl.pallas_call(
        _add_body,
        grid=(N_PAD // bn,),
        in_specs=[pl.BlockSpec((NC, bn, D_FEAT), lambda i: (0, i, 0))],
        out_specs=pl.BlockSpec((bn, D_FEAT), lambda i: (i, 0)),
        out_shape=jax.ShapeDtypeStruct((N_PAD, D_FEAT), jnp.float32),
    )(p)


def kernel(inputs, edge_index, edge_weight):
    row = edge_index[0].astype(jnp.int32)
    col = edge_index[1].astype(jnp.int32)
    w = edge_weight.astype(jnp.float32)
    pad = E_PAD - row.shape[0]
    row = jnp.pad(row, (0, pad)).reshape(NW, CHUNKS_PER_W, CHUNK)
    col = jnp.pad(col, (0, pad)).reshape(NW, CHUNKS_PER_W, CHUNK)
    w = jnp.pad(w, (0, pad)).reshape(NW, CHUNKS_PER_W, CHUNK)
    partials = _sc_spmm(inputs, row, col, w)
    return _sum_partials(partials)[:N_NODES]


# speedup vs baseline: 3.6591x; 1.1941x over previous
import functools

import jax
import jax.numpy as jnp
from jax import lax
from jax.experimental import pallas as pl
from jax.experimental.pallas import tpu as pltpu
from jax.experimental.pallas import tpu_sc as plsc

N_NODES = 10000
D_FEAT = 128
N_EDGES = 320000

NC = 2
NS = 16
NW = NC * NS
CHUNK = 128
N_SEC = 2
SEC_CHUNKS = 40
CHUNKS_PER_W = N_SEC * SEC_CHUNKS
EDGES_PER_W = CHUNK * CHUNKS_PER_W
E_PAD = NW * EDGES_PER_W
N_PAD = 10112
ROWS_PER_W = N_PAD // NS
DL = D_FEAT // 16


def _sc_body(x_hbm, row_hbm, col_hbm, w_hbm, out_hbm,
             row_all, col_all, w_all, rows_a, rows_b, shared,
             sem_g0, sem_g1):
    c = lax.axis_index("c")
    s = lax.axis_index("s")
    wid = s * NC + c

    def zero_body(e, _):
        for d in range(DL):
            rows_a[e, pl.ds(d * 16, 16)] = jnp.zeros((16,), jnp.float32)
        return 0
    lax.fori_loop(0, CHUNK, zero_body, 0)
    base = s * ROWS_PER_W
    nz = ROWS_PER_W // CHUNK
    for k in range(nz):
        pltpu.sync_copy(rows_a, shared.at[pl.ds(base + k * CHUNK, CHUNK)])
    if ROWS_PER_W % CHUNK:
        pltpu.sync_copy(rows_a.at[pl.ds(0, ROWS_PER_W % CHUNK)],
                        shared.at[pl.ds(base + nz * CHUNK, ROWS_PER_W % CHUNK)])

    bufs = (rows_a, rows_b)
    gsems = (sem_g0, sem_g1)

    def gdesc(j, b):
        return pltpu.make_async_copy(x_hbm.at[row_all.at[j]], bufs[b], gsems[b])

    def scale(j, buf):
        def mul_body(g, _2):
            wv = w_all[j, pl.ds(g * 16, 16)]
            wv = jnp.where((wv - wv) == 0.0, wv, 0.0)
            for l in range(16):
                w = wv[l]
                e = g * 16 + l
                for d in range(DL):
                    sl = buf[e, pl.ds(d * 16, 16)]
                    buf[e, pl.ds(d * 16, 16)] = sl * w
            return 0
        lax.fori_loop(0, CHUNK // 16, mul_body, 0)

    for sec in range(N_SEC):
        sbase = sec * SEC_CHUNKS
        pltpu.sync_copy(row_hbm.at[wid, pl.ds(sbase, SEC_CHUNKS)], row_all)
        pltpu.sync_copy(col_hbm.at[wid, pl.ds(sbase, SEC_CHUNKS)], col_all)
        pltpu.sync_copy(w_hbm.at[wid, pl.ds(sbase, SEC_CHUNKS)], w_all)

        gdesc(0, 0).start()

        def step(j, b):
            nb = 1 - b

            @pl.when(j < SEC_CHUNKS - 1)
            def _next_gather():
                gdesc(j + 1, nb).start()

            gdesc(0, b).wait()
            scale(j, bufs[b])
            pltpu.sync_copy(bufs[b], shared.at[col_all.at[j]], add=True)

        def outer(g, _):
            for b in range(2):
                step(2 * g + b, b)
            return 0
        lax.fori_loop(0, SEC_CHUNKS // 2, outer, 0)

    plsc.subcore_barrier()

    pltpu.sync_copy(shared.at[pl.ds(base, ROWS_PER_W)],
                    out_hbm.at[c, pl.ds(base, ROWS_PER_W)])


_sc_spmm = functools.partial(
    pl.kernel,
    out_type=jax.ShapeDtypeStruct((NC, N_PAD, D_FEAT), jnp.float32),
    mesh=plsc.VectorSubcoreMesh(core_axis_name="c", subcore_axis_name="s"),
    scratch_types=[
        pltpu.VMEM((SEC_CHUNKS, CHUNK), jnp.int32),
        pltpu.VMEM((SEC_CHUNKS, CHUNK), jnp.int32),
        pltpu.VMEM((SEC_CHUNKS, CHUNK), jnp.float32),
        pltpu.VMEM((CHUNK, D_FEAT), jnp.float32),
        pltpu.VMEM((CHUNK, D_FEAT), jnp.float32),
        pltpu.VMEM_SHARED((N_PAD, D_FEAT), jnp.float32),
        pltpu.SemaphoreType.DMA,
        pltpu.SemaphoreType.DMA,
    ],
)(_sc_body)


def _add_body(p_ref, o_ref):
    o_ref[...] = p_ref[0] + p_ref[1]


def _sum_partials(p):
    bn = ROWS_PER_W
    return pl.pallas_call(
        _add_body,
        grid=(N_PAD // bn,),
        in_specs=[pl.BlockSpec((NC, bn, D_FEAT), lambda i: (0, i, 0))],
        out_specs=pl.BlockSpec((bn, D_FEAT), lambda i: (i, 0)),
        out_shape=jax.ShapeDtypeStruct((N_PAD, D_FEAT), jnp.float32),
    )(p)


def kernel(inputs, edge_index, edge_weight):
    row = edge_index[0].astype(jnp.int32)
    col = edge_index[1].astype(jnp.int32)
    w = edge_weight.astype(jnp.float32)
    pad = E_PAD - row.shape[0]
    row = jnp.pad(row, (0, pad)).reshape(NW, CHUNKS_PER_W, CHUNK)
    col = jnp.pad(col, (0, pad)).reshape(NW, CHUNKS_PER_W, CHUNK)
    w = jnp.pad(w, (0, pad)).reshape(NW, CHUNKS_PER_W, CHUNK)
    partials = _sc_spmm(inputs, row, col, w)
    return _sum_partials(partials)[:N_NODES]


# speedup vs baseline: 4.2012x; 1.1481x over previous
import functools

import jax
import jax.numpy as jnp
from jax import lax
from jax.experimental import pallas as pl
from jax.experimental.pallas import tpu as pltpu
from jax.experimental.pallas import tpu_sc as plsc

N_NODES = 10000
D_FEAT = 128
N_EDGES = 320000

NC = 2
NS = 16
NW = NC * NS
CHUNK = 128
SEC_CHUNKS = 40
FAST_C = 1
FAST_SEC = 3
SLOW_SEC = 1
FAST_CHUNKS = FAST_SEC * SEC_CHUNKS
SLOW_CHUNKS = SLOW_SEC * SEC_CHUNKS
TOT_CHUNKS = NS * (FAST_CHUNKS + SLOW_CHUNKS)
E_PAD = TOT_CHUNKS * CHUNK
N_PAD = 10112
ROWS_PER_W = N_PAD // NS
DL = D_FEAT // 16


def _sc_body(x_hbm, row_hbm, col_hbm, w_hbm, out_hbm,
             row_all, col_all, w_all, rows_a, rows_b, shared,
             sem_g0, sem_g1):
    c = lax.axis_index("c")
    s = lax.axis_index("s")

    def zero_body(e, _):
        for d in range(DL):
            rows_a[e, pl.ds(d * 16, 16)] = jnp.zeros((16,), jnp.float32)
        return 0
    lax.fori_loop(0, CHUNK, zero_body, 0)
    base = s * ROWS_PER_W
    nz = ROWS_PER_W // CHUNK
    for k in range(nz):
        pltpu.sync_copy(rows_a, shared.at[pl.ds(base + k * CHUNK, CHUNK)])
    if ROWS_PER_W % CHUNK:
        pltpu.sync_copy(rows_a.at[pl.ds(0, ROWS_PER_W % CHUNK)],
                        shared.at[pl.ds(base + nz * CHUNK, ROWS_PER_W % CHUNK)])

    bufs = (rows_a, rows_b)
    gsems = (sem_g0, sem_g1)

    def gdesc(j, b):
        return pltpu.make_async_copy(x_hbm.at[row_all.at[j]], bufs[b], gsems[b])

    def scale(j, buf):
        def mul_body(g, _2):
            wv = w_all[j, pl.ds(g * 16, 16)]
            wv = jnp.where((wv - wv) == 0.0, wv, 0.0)
            for l in range(16):
                w = wv[l]
                e = g * 16 + l
                for d in range(DL):
                    sl = buf[e, pl.ds(d * 16, 16)]
                    buf[e, pl.ds(d * 16, 16)] = sl * w
            return 0
        lax.fori_loop(0, CHUNK // 16, mul_body, 0)

    def run_section(sbase):
        pltpu.sync_copy(row_hbm.at[pl.ds(sbase, SEC_CHUNKS)], row_all)
        pltpu.sync_copy(col_hbm.at[pl.ds(sbase, SEC_CHUNKS)], col_all)
        pltpu.sync_copy(w_hbm.at[pl.ds(sbase, SEC_CHUNKS)], w_all)

        gdesc(0, 0).start()

        def step(j, b):
            nb = 1 - b

            @pl.when(j < SEC_CHUNKS - 1)
            def _next_gather():
                gdesc(j + 1, nb).start()

            gdesc(0, b).wait()
            scale(j, bufs[b])
            pltpu.sync_copy(bufs[b], shared.at[col_all.at[j]], add=True)

        def outer(g, _):
            for b in range(2):
                step(2 * g + b, b)
            return 0
        lax.fori_loop(0, SEC_CHUNKS // 2, outer, 0)

    @pl.when(c == FAST_C)
    def _fast():
        my_base = s * FAST_CHUNKS
        for sec in range(FAST_SEC):
            run_section(my_base + sec * SEC_CHUNKS)

    @pl.when(c != FAST_C)
    def _slow():
        my_base = NS * FAST_CHUNKS + s * SLOW_CHUNKS
        for sec in range(SLOW_SEC):
            run_section(my_base + sec * SEC_CHUNKS)

    plsc.subcore_barrier()

    pltpu.sync_copy(shared.at[pl.ds(base, ROWS_PER_W)],
                    out_hbm.at[c, pl.ds(base, ROWS_PER_W)])


_sc_spmm = functools.partial(
    pl.kernel,
    out_type=jax.ShapeDtypeStruct((NC, N_PAD, D_FEAT), jnp.float32),
    mesh=plsc.VectorSubcoreMesh(core_axis_name="c", subcore_axis_name="s"),
    scratch_types=[
        pltpu.VMEM((SEC_CHUNKS, CHUNK), jnp.int32),
        pltpu.VMEM((SEC_CHUNKS, CHUNK), jnp.int32),
        pltpu.VMEM((SEC_CHUNKS, CHUNK), jnp.float32),
        pltpu.VMEM((CHUNK, D_FEAT), jnp.float32),
        pltpu.VMEM((CHUNK, D_FEAT), jnp.float32),
        pltpu.VMEM_SHARED((N_PAD, D_FEAT), jnp.float32),
        pltpu.SemaphoreType.DMA,
        pltpu.SemaphoreType.DMA,
    ],
)(_sc_body)


def _add_body(p_ref, o_ref):
    o_ref[...] = p_ref[0] + p_ref[1]


def _sum_partials(p):
    bn = ROWS_PER_W
    return pl.pallas_call(
        _add_body,
        grid=(N_PAD // bn,),
        in_specs=[pl.BlockSpec((NC, bn, D_FEAT), lambda i: (0, i, 0))],
        out_specs=pl.BlockSpec((bn, D_FEAT), lambda i: (i, 0)),
        out_shape=jax.ShapeDtypeStruct((N_PAD, D_FEAT), jnp.float32),
    )(p)


def kernel(inputs, edge_index, edge_weight):
    row = edge_index[0].astype(jnp.int32)
    col = edge_index[1].astype(jnp.int32)
    w = edge_weight.astype(jnp.float32)
    pad = E_PAD - row.shape[0]
    row = jnp.pad(row, (0, pad)).reshape(TOT_CHUNKS, CHUNK)
    col = jnp.pad(col, (0, pad)).reshape(TOT_CHUNKS, CHUNK)
    w = jnp.pad(w, (0, pad)).reshape(TOT_CHUNKS, CHUNK)
    partials = _sc_spmm(inputs, row, col, w)
    return _sum_partials(partials)[:N_NODES]
